# Initial kernel scaffold; baseline (speedup 1.0000x reference)
import jax
import jax.numpy as jnp
from jax.experimental import pallas as pl


def kernel(tables, values):
    raise NotImplementedError("write your pallas kernel here")


# speedup vs baseline: 1.1223x; 1.1223x over previous
import functools

import jax
import jax.numpy as jnp
from jax import lax
from jax.experimental import pallas as pl
from jax.experimental.pallas import tpu as pltpu
from jax.experimental.pallas import tpu_sc as plsc

F = 26
V = 100000
D = 32
B = 16384
NC = 2
NS = 16
NW = NC * NS
BPW = B // NW
RPC = 4
IDXC = RPC * F
CPW = BPW // RPC
L = 16


def _sc_encode():
    mesh = plsc.VectorSubcoreMesh(core_axis_name="c", subcore_axis_name="s")

    @functools.partial(
        pl.kernel,
        out_type=jax.ShapeDtypeStruct((B, D), jnp.float32),
        mesh=mesh,
        scratch_types=[
            pltpu.VMEM((CPW, IDXC), jnp.int32),
            pltpu.VMEM((IDXC, D), jnp.float32),
            pltpu.VMEM((BPW, D), jnp.float32),
            pltpu.SemaphoreType.DMA,
        ],
        compiler_params=pltpu.CompilerParams(use_tc_tiling_on_sc=False),
    )
    def body(tbl_hbm, idx_hbm, out_hbm, idx_v, gbuf, obuf, sem):
        wid = lax.axis_index("s") * NC + lax.axis_index("c")
        pltpu.sync_copy(idx_hbm.at[wid], idx_v)

        def chunk(c, carry):
            pltpu.async_copy(tbl_hbm.at[idx_v.at[c]], gbuf, sem).wait()
            for b in range(RPC):
                acc0 = gbuf[b * F, pl.ds(0, L)]
                acc1 = gbuf[b * F, pl.ds(L, L)]
                for f in range(1, F):
                    r = b * F + f
                    acc0 = acc0 + gbuf[r, pl.ds(0, L)]
                    acc1 = acc1 + gbuf[r, pl.ds(L, L)]
                obuf[c * RPC + b, pl.ds(0, L)] = acc0
                obuf[c * RPC + b, pl.ds(L, L)] = acc1
            return carry

        lax.fori_loop(0, CPW, chunk, 0)
        pltpu.sync_copy(obuf, out_hbm.at[pl.ds(wid * BPW, BPW)])

    return body


def kernel(tables, values):
    tables_flat = tables.reshape(F * V, D)
    offs = (jnp.arange(F, dtype=jnp.int32) * V)[None, :]
    flat_idx = values.astype(jnp.int32) + offs
    idx_prep = flat_idx.reshape(NW, CPW, IDXC)
    return _sc_encode()(tables_flat, idx_prep)


# speedup vs baseline: 1.1941x; 1.0640x over previous
import functools

import jax
import jax.numpy as jnp
from jax import lax
from jax.experimental import pallas as pl
from jax.experimental.pallas import tpu as pltpu
from jax.experimental.pallas import tpu_sc as plsc

F = 26
V = 100000
D = 32
B = 16384
NC = 2
NS = 16
NW = NC * NS
BPW = B // NW
RPC = 4
IDXC = RPC * F
CPW = BPW // RPC
L = 16


NBUF = 4


def _sc_encode():
    mesh = plsc.VectorSubcoreMesh(core_axis_name="c", subcore_axis_name="s")

    @functools.partial(
        pl.kernel,
        out_type=jax.ShapeDtypeStruct((B, D), jnp.float32),
        mesh=mesh,
        scratch_types=[
            pltpu.VMEM((CPW, IDXC), jnp.int32),
            pltpu.VMEM((NBUF, IDXC, D), jnp.float32),
            pltpu.VMEM((BPW, D), jnp.float32),
            pltpu.SemaphoreType.DMA((NBUF,)),
        ],
        compiler_params=pltpu.CompilerParams(use_tc_tiling_on_sc=False),
    )
    def body(tbl_hbm, idx_hbm, out_hbm, idx_v, gbuf, obuf, sems):
        wid = lax.axis_index("s") * NC + lax.axis_index("c")
        pltpu.sync_copy(idx_hbm.at[wid], idx_v)

        for b in range(NBUF):
            pltpu.async_copy(tbl_hbm.at[idx_v.at[b]], gbuf.at[b], sems.at[b])

        def group(i, carry):
            c = i * NBUF
            for b in range(NBUF):
                cid = c + b
                pltpu.make_async_copy(
                    tbl_hbm.at[idx_v.at[cid]], gbuf.at[b], sems.at[b]
                ).wait()
                for r in range(RPC):
                    acc0 = gbuf[b, r * F, pl.ds(0, L)]
                    acc1 = gbuf[b, r * F, pl.ds(L, L)]
                    for f in range(1, F):
                        row = r * F + f
                        acc0 = acc0 + gbuf[b, row, pl.ds(0, L)]
                        acc1 = acc1 + gbuf[b, row, pl.ds(L, L)]
                    obuf[cid * RPC + r, pl.ds(0, L)] = acc0
                    obuf[cid * RPC + r, pl.ds(L, L)] = acc1
                nxt = cid + NBUF

                @pl.when(nxt < CPW)
                def _():
                    pltpu.async_copy(
                        tbl_hbm.at[idx_v.at[nxt]], gbuf.at[b], sems.at[b]
                    )

            return carry

        lax.fori_loop(0, CPW // NBUF, group, 0)
        pltpu.sync_copy(obuf, out_hbm.at[pl.ds(wid * BPW, BPW)])

    return body


def kernel(tables, values):
    tables_flat = tables.reshape(F * V, D)
    offs = (jnp.arange(F, dtype=jnp.int32) * V)[None, :]
    flat_idx = values.astype(jnp.int32) + offs
    idx_prep = flat_idx.reshape(NW, CPW, IDXC)
    return _sc_encode()(tables_flat, idx_prep)
